# Initial kernel scaffold; baseline (speedup 1.0000x reference)
import jax
import jax.numpy as jnp
from jax.experimental import pallas as pl


def kernel(token_embedding_weight, ctx, tokenized_prompts):
    raise NotImplementedError("write your pallas kernel here")


# speedup vs baseline: 1.1057x; 1.1057x over previous
import functools

import jax
import jax.numpy as jnp
from jax import lax
from jax.experimental import pallas as pl
from jax.experimental.pallas import tpu as pltpu
from jax.experimental.pallas import tpu_sc as plsc

N_CLS = 1000
CTX_LEN = 77
D = 768
N_CTX = 8
N_SUF = CTX_LEN - 1 - N_CTX
ROW_PAD = 72
NC = 2
NS = 16
NW = NC * NS
CPW = (N_CLS + NW - 1) // NW
N_CLS_PAD = NW * CPW


_mesh = plsc.VectorSubcoreMesh(core_axis_name="c", subcore_axis_name="s")


@functools.partial(
    pl.kernel,
    out_type=jax.ShapeDtypeStruct((N_CLS * CTX_LEN, D), jnp.float32),
    mesh=_mesh,
    scratch_types=[
        pltpu.VMEM((CPW, ROW_PAD), jnp.int32),
        pltpu.VMEM((ROW_PAD, D), jnp.float32),
        pltpu.VMEM((N_CTX, D), jnp.float32),
        pltpu.SemaphoreType.DMA,
    ],
    compiler_params=pltpu.CompilerParams(use_tc_tiling_on_sc=False),
)
def _prompt_gather(table, ctx, idx, out, idx_v, buf, ctx_v, sem):
    w = lax.axis_index("s") * NC + lax.axis_index("c")
    c0 = w * CPW
    pltpu.sync_copy(ctx, ctx_v)
    pltpu.sync_copy(idx.at[pl.ds(c0, CPW)], idx_v)

    def body(i, carry):
        c = c0 + i

        @pl.when(c < N_CLS)
        def _():
            pltpu.async_copy(table.at[idx_v.at[i]], buf, sem).wait()
            base = c * CTX_LEN
            pltpu.sync_copy(buf.at[pl.ds(0, 1)], out.at[pl.ds(base, 1)])
            pltpu.sync_copy(ctx_v, out.at[pl.ds(base + 1, N_CTX)])
            pltpu.sync_copy(
                buf.at[pl.ds(1, N_SUF)],
                out.at[pl.ds(base + 1 + N_CTX, N_SUF)],
            )

        return carry

    lax.fori_loop(0, CPW, body, 0)


def kernel(token_embedding_weight, ctx, tokenized_prompts):
    idx = jnp.concatenate(
        [
            tokenized_prompts[:, :1],
            tokenized_prompts[:, 1 + N_CTX:],
            jnp.zeros((N_CLS, ROW_PAD - CTX_LEN + N_CTX), jnp.int32),
        ],
        axis=1,
    )
    idx = jnp.pad(idx, ((0, N_CLS_PAD - N_CLS), (0, 0)))
    out = _prompt_gather(token_embedding_weight, ctx, idx)
    return out.reshape(N_CLS, CTX_LEN, D)


# speedup vs baseline: 1.1081x; 1.0022x over previous
import functools

import jax
import jax.numpy as jnp
from jax import lax
from jax.experimental import pallas as pl
from jax.experimental.pallas import tpu as pltpu
from jax.experimental.pallas import tpu_sc as plsc

N_CLS = 1000
CTX_LEN = 77
D = 768
N_CTX = 8
N_SUF = CTX_LEN - 1 - N_CTX
ROW_PAD = 72
NC = 2
NS = 16
NW = NC * NS
CPW = (N_CLS + NW - 1) // NW
N_CLS_PAD = NW * CPW


_mesh = plsc.VectorSubcoreMesh(core_axis_name="c", subcore_axis_name="s")


@functools.partial(
    pl.kernel,
    out_type=jax.ShapeDtypeStruct((N_CLS * CTX_LEN, D), jnp.float32),
    mesh=_mesh,
    scratch_types=[
        pltpu.VMEM((CPW, ROW_PAD), jnp.int32),
        pltpu.VMEM((2, ROW_PAD, D), jnp.float32),
        pltpu.VMEM((N_CTX, D), jnp.float32),
        pltpu.SemaphoreType.DMA,
    ],
    compiler_params=pltpu.CompilerParams(use_tc_tiling_on_sc=False),
)
def _prompt_gather(table, ctx, idx, out, idx_v, buf, ctx_v, gsem):
    w = lax.axis_index("s") * NC + lax.axis_index("c")
    c0 = w * CPW
    n_w = jnp.minimum(CPW, N_CLS - c0)
    pltpu.sync_copy(ctx, ctx_v)
    pltpu.sync_copy(idx.at[pl.ds(c0, CPW)], idx_v)

    def _start_gather(i):
        pltpu.async_copy(table.at[idx_v.at[i]], buf.at[i % 2], gsem)

    def body(i, carry):
        c = c0 + i

        @pl.when(i < n_w)
        def _():
            @pl.when(i == 0)
            def _():
                _start_gather(i)

            pltpu.make_async_copy(
                table.at[idx_v.at[i]], buf.at[i % 2], gsem
            ).wait()

            @pl.when(i + 1 < n_w)
            def _():
                _start_gather(i + 1)

            b = buf.at[i % 2]
            base = c * CTX_LEN
            pltpu.sync_copy(b.at[pl.ds(0, 1)], out.at[pl.ds(base, 1)])
            pltpu.sync_copy(ctx_v, out.at[pl.ds(base + 1, N_CTX)])
            pltpu.sync_copy(
                b.at[pl.ds(1, N_SUF)],
                out.at[pl.ds(base + 1 + N_CTX, N_SUF)],
            )

        return carry

    lax.fori_loop(0, CPW, body, 0)


def kernel(token_embedding_weight, ctx, tokenized_prompts):
    idx = jnp.concatenate(
        [
            tokenized_prompts[:, :1],
            tokenized_prompts[:, 1 + N_CTX:],
            jnp.zeros((N_CLS, ROW_PAD - CTX_LEN + N_CTX), jnp.int32),
        ],
        axis=1,
    )
    idx = jnp.pad(idx, ((0, N_CLS_PAD - N_CLS), (0, 0)))
    out = _prompt_gather(token_embedding_weight, ctx, idx)
    return out.reshape(N_CLS, CTX_LEN, D)


# speedup vs baseline: 4.2413x; 3.8277x over previous
import functools

import jax
import jax.numpy as jnp
from jax import lax
from jax.experimental import pallas as pl
from jax.experimental.pallas import tpu as pltpu
from jax.experimental.pallas import tpu_sc as plsc

N_CLS = 1000
CTX_LEN = 77
VOCAB = 49408
D = 768
N_CTX = 8
LANES = 128
DB = D // LANES
CPB = 40
KC = N_CLS // CPB
UNIT = CPB * DB
HALF = UNIT // 2
UNITS = CTX_LEN * KC
NW = 32
UPW = (UNITS + NW - 1) // NW
UNITS_PAD = NW * UPW
NBUF = 2

_mesh = plsc.VectorSubcoreMesh(core_axis_name="c", subcore_axis_name="s")


@functools.partial(
    pl.kernel,
    out_type=jax.ShapeDtypeStruct((CTX_LEN * KC * UNIT, LANES), jnp.float32),
    mesh=_mesh,
    scratch_types=[
        pltpu.VMEM((UPW, 2, HALF), jnp.int32),
        pltpu.VMEM((NBUF, UNIT, LANES), jnp.float32),
        pltpu.SemaphoreType.DMA,
    ],
    compiler_params=pltpu.CompilerParams(use_tc_tiling_on_sc=False),
)
def _prompt_gather(table_c, ctx_c, gidx, out, gidx_v, bufs, gsem):
    w = lax.axis_index("s") * 2 + lax.axis_index("c")
    u0 = w * UPW
    n_u = jnp.minimum(UPW, UNITS - u0)
    pltpu.sync_copy(gidx.at[pl.ds(u0, UPW)], gidx_v)

    def unit_t(j):
        return (u0 + j) // KC

    def is_ctx(t):
        return (t >= 1) & (t <= 1 + N_CTX - 1)

    def start_gathers(j):
        b = j % NBUF
        t = unit_t(j)

        @pl.when(is_ctx(t))
        def _():
            pltpu.async_copy(
                ctx_c.at[gidx_v.at[j, 0]], bufs.at[b, pl.ds(0, HALF)], gsem)
            pltpu.async_copy(
                ctx_c.at[gidx_v.at[j, 1]], bufs.at[b, pl.ds(HALF, HALF)], gsem)

        @pl.when(~is_ctx(t))
        def _():
            pltpu.async_copy(
                table_c.at[gidx_v.at[j, 0]], bufs.at[b, pl.ds(0, HALF)], gsem)
            pltpu.async_copy(
                table_c.at[gidx_v.at[j, 1]], bufs.at[b, pl.ds(HALF, HALF)], gsem)

    def wait_gathers(j):
        b = j % NBUF
        t = unit_t(j)

        @pl.when(is_ctx(t))
        def _():
            pltpu.make_async_copy(
                ctx_c.at[gidx_v.at[j, 0]], bufs.at[b, pl.ds(0, HALF)], gsem
            ).wait()
            pltpu.make_async_copy(
                ctx_c.at[gidx_v.at[j, 1]], bufs.at[b, pl.ds(HALF, HALF)], gsem
            ).wait()

        @pl.when(~is_ctx(t))
        def _():
            pltpu.make_async_copy(
                table_c.at[gidx_v.at[j, 0]], bufs.at[b, pl.ds(0, HALF)], gsem
            ).wait()
            pltpu.make_async_copy(
                table_c.at[gidx_v.at[j, 1]], bufs.at[b, pl.ds(HALF, HALF)], gsem
            ).wait()

    def body(j, carry):
        @pl.when(j < n_u)
        def _():
            @pl.when(j == 0)
            def _():
                start_gathers(0)

            wait_gathers(j)

            @pl.when(j + 1 < n_u)
            def _():
                start_gathers(j + 1)

            u = u0 + j
            pltpu.sync_copy(
                bufs.at[j % NBUF], out.at[pl.ds(u * UNIT, UNIT)])

        return carry

    lax.fori_loop(0, UPW, body, 0)


def kernel(token_embedding_weight, ctx, tokenized_prompts):
    table_c = (
        token_embedding_weight.reshape(VOCAB // 8, 8, DB, LANES)
        .transpose(0, 2, 1, 3)
        .reshape(VOCAB * DB, LANES)
    )
    ctx_c = (
        ctx.reshape(1, N_CTX, DB, LANES)
        .transpose(0, 2, 1, 3)
        .reshape(N_CTX * DB, LANES)
    )

    rt = tokenized_prompts.T.reshape(CTX_LEN, KC, CPB // 8, 8)
    base = (rt // 8) * (8 * DB) + (rt % 8)
    dmul = (jnp.arange(DB, dtype=jnp.int32) * 8)[None, None, None, :, None]
    gidx = (base[:, :, :, None, :] + dmul).reshape(CTX_LEN, KC, UNIT)
    cpat = (
        jnp.arange(N_CTX, dtype=jnp.int32)[:, None, None, None]
        + (jnp.arange(DB, dtype=jnp.int32) * 8)[None, None, :, None]
        + jnp.zeros((N_CTX, CPB // 8, DB, 8), jnp.int32)
    ).reshape(N_CTX, UNIT)
    gidx = gidx.at[1:1 + N_CTX].set(cpat[:, None, :])
    gidx = gidx.reshape(UNITS, 2, HALF)
    gidx = jnp.pad(gidx, ((0, UNITS_PAD - UNITS), (0, 0), (0, 0)))

    out = _prompt_gather(table_c, ctx_c, gidx)
    return (
        out.reshape(CTX_LEN, N_CLS // 8, DB, 8, LANES)
        .transpose(1, 3, 0, 2, 4)
        .reshape(N_CLS, CTX_LEN, D)
    )


# speedup vs baseline: 4.2478x; 1.0015x over previous
import functools

import jax
import jax.numpy as jnp
from jax import lax
from jax.experimental import pallas as pl
from jax.experimental.pallas import tpu as pltpu
from jax.experimental.pallas import tpu_sc as plsc

N_CLS = 1000
CTX_LEN = 77
VOCAB = 49408
D = 768
N_CTX = 8
LANES = 128
DB = D // LANES
CPB = 40
KC = N_CLS // CPB
UNIT = CPB * DB
HALF = UNIT // 2
UNITS = CTX_LEN * KC
NW = 32
UPW = (UNITS + NW - 1) // NW
UNITS_PAD = NW * UPW
NBUF = 3

_mesh = plsc.VectorSubcoreMesh(core_axis_name="c", subcore_axis_name="s")


@functools.partial(
    pl.kernel,
    out_type=jax.ShapeDtypeStruct((CTX_LEN * KC * UNIT, LANES), jnp.float32),
    mesh=_mesh,
    scratch_types=[
        pltpu.VMEM((UPW, 2, HALF), jnp.int32),
        pltpu.VMEM((NBUF, UNIT, LANES), jnp.float32),
        pltpu.SemaphoreType.DMA,
        pltpu.SemaphoreType.DMA,
    ],
    compiler_params=pltpu.CompilerParams(use_tc_tiling_on_sc=False),
)
def _prompt_gather(table_c, ctx_c, gidx, out, gidx_v, bufs, gsem, wsem):
    w = lax.axis_index("s") * 2 + lax.axis_index("c")
    u0 = w * UPW
    n_u = jnp.minimum(UPW, UNITS - u0)
    pltpu.sync_copy(gidx.at[pl.ds(u0, UPW)], gidx_v)

    def unit_t(j):
        return (u0 + j) // KC

    def is_ctx(t):
        return (t >= 1) & (t <= 1 + N_CTX - 1)

    def start_gathers(j):
        b = j % NBUF
        t = unit_t(j)

        @pl.when(is_ctx(t))
        def _():
            pltpu.async_copy(
                ctx_c.at[gidx_v.at[j, 0]], bufs.at[b, pl.ds(0, HALF)], gsem)
            pltpu.async_copy(
                ctx_c.at[gidx_v.at[j, 1]], bufs.at[b, pl.ds(HALF, HALF)], gsem)

        @pl.when(~is_ctx(t))
        def _():
            pltpu.async_copy(
                table_c.at[gidx_v.at[j, 0]], bufs.at[b, pl.ds(0, HALF)], gsem)
            pltpu.async_copy(
                table_c.at[gidx_v.at[j, 1]], bufs.at[b, pl.ds(HALF, HALF)], gsem)

    def wait_gathers(j):
        b = j % NBUF
        t = unit_t(j)

        @pl.when(is_ctx(t))
        def _():
            pltpu.make_async_copy(
                ctx_c.at[gidx_v.at[j, 0]], bufs.at[b, pl.ds(0, HALF)], gsem
            ).wait()
            pltpu.make_async_copy(
                ctx_c.at[gidx_v.at[j, 1]], bufs.at[b, pl.ds(HALF, HALF)], gsem
            ).wait()

        @pl.when(~is_ctx(t))
        def _():
            pltpu.make_async_copy(
                table_c.at[gidx_v.at[j, 0]], bufs.at[b, pl.ds(0, HALF)], gsem
            ).wait()
            pltpu.make_async_copy(
                table_c.at[gidx_v.at[j, 1]], bufs.at[b, pl.ds(HALF, HALF)], gsem
            ).wait()

    def wait_write(j):
        pltpu.make_async_copy(
            bufs.at[j % NBUF], out.at[pl.ds((u0 + j) * UNIT, UNIT)], wsem
        ).wait()

    def body(j, carry):
        @pl.when(j < n_u)
        def _():
            @pl.when(j == 0)
            def _():
                start_gathers(0)

            wait_gathers(j)

            @pl.when(j + 1 < n_u)
            def _():
                @pl.when(j + 1 >= NBUF)
                def _():
                    wait_write(j + 1 - NBUF)

                start_gathers(j + 1)

            pltpu.async_copy(
                bufs.at[j % NBUF],
                out.at[pl.ds((u0 + j) * UNIT, UNIT)],
                wsem,
            )

        return carry

    lax.fori_loop(0, UPW, body, 0)

    def drain(m, carry):
        j = n_u - NBUF + m

        @pl.when(j >= 0)
        def _():
            wait_write(j)

        return carry

    lax.fori_loop(0, NBUF, drain, 0)


def kernel(token_embedding_weight, ctx, tokenized_prompts):
    table_c = (
        token_embedding_weight.reshape(VOCAB // 8, 8, DB, LANES)
        .transpose(0, 2, 1, 3)
        .reshape(VOCAB * DB, LANES)
    )
    ctx_c = (
        ctx.reshape(1, N_CTX, DB, LANES)
        .transpose(0, 2, 1, 3)
        .reshape(N_CTX * DB, LANES)
    )

    rt = tokenized_prompts.T.reshape(CTX_LEN, KC, CPB // 8, 8)
    base = (rt // 8) * (8 * DB) + (rt % 8)
    dmul = (jnp.arange(DB, dtype=jnp.int32) * 8)[None, None, None, :, None]
    gidx = (base[:, :, :, None, :] + dmul).reshape(CTX_LEN, KC, UNIT)
    cpat = (
        jnp.arange(N_CTX, dtype=jnp.int32)[:, None, None, None]
        + (jnp.arange(DB, dtype=jnp.int32) * 8)[None, None, :, None]
        + jnp.zeros((N_CTX, CPB // 8, DB, 8), jnp.int32)
    ).reshape(N_CTX, UNIT)
    gidx = gidx.at[1:1 + N_CTX].set(cpat[:, None, :])
    gidx = gidx.reshape(UNITS, 2, HALF)
    gidx = jnp.pad(gidx, ((0, UNITS_PAD - UNITS), (0, 0), (0, 0)))

    out = _prompt_gather(table_c, ctx_c, gidx)
    return (
        out.reshape(CTX_LEN, N_CLS // 8, DB, 8, LANES)
        .transpose(1, 3, 0, 2, 4)
        .reshape(N_CLS, CTX_LEN, D)
    )


# speedup vs baseline: 4.5427x; 1.0694x over previous
import functools

import jax
import jax.numpy as jnp
from jax import lax
from jax.experimental import pallas as pl
from jax.experimental.pallas import tpu as pltpu
from jax.experimental.pallas import tpu_sc as plsc

N_CLS = 1000
CTX_LEN = 77
VOCAB = 49408
D = 768
N_CTX = 8
LANES = 128
DB = D // LANES
CPB = 40
KC = N_CLS // CPB
UNIT = CPB * DB
HALF = UNIT // 2
UNITS = CTX_LEN * KC
NW = 32
UPW = (UNITS + NW - 1) // NW
UNITS_PAD = NW * UPW
NBUF = 3

_mesh = plsc.VectorSubcoreMesh(core_axis_name="c", subcore_axis_name="s")


@functools.partial(
    pl.kernel,
    out_type=jax.ShapeDtypeStruct((CTX_LEN * KC * UNIT, LANES), jnp.float32),
    mesh=_mesh,
    scratch_types=[
        pltpu.VMEM((UPW, 2, HALF), jnp.int32),
        pltpu.VMEM((NBUF, UNIT, LANES), jnp.float32),
        pltpu.SemaphoreType.DMA,
        pltpu.SemaphoreType.DMA,
    ],
    compiler_params=pltpu.CompilerParams(use_tc_tiling_on_sc=False),
)
def _prompt_gather(table_c, ctx_c, gidx, out, gidx_v, bufs, gsem, wsem):
    w = lax.axis_index("s") * 2 + lax.axis_index("c")
    u0 = w * UPW
    n_u = jnp.minimum(UPW, UNITS - u0)
    pltpu.sync_copy(gidx.at[pl.ds(u0, UPW)], gidx_v)

    def unit_t(j):
        return (u0 + j) // KC

    def is_ctx(t):
        return (t >= 1) & (t <= 1 + N_CTX - 1)

    def start_gathers(j):
        b = j % NBUF
        t = unit_t(j)

        @pl.when(is_ctx(t))
        def _():
            pltpu.async_copy(
                ctx_c.at[gidx_v.at[j, 0]], bufs.at[b, pl.ds(0, HALF)], gsem)
            pltpu.async_copy(
                ctx_c.at[gidx_v.at[j, 1]], bufs.at[b, pl.ds(HALF, HALF)], gsem)

        @pl.when(~is_ctx(t))
        def _():
            pltpu.async_copy(
                table_c.at[gidx_v.at[j, 0]], bufs.at[b, pl.ds(0, HALF)], gsem)
            pltpu.async_copy(
                table_c.at[gidx_v.at[j, 1]], bufs.at[b, pl.ds(HALF, HALF)], gsem)

    def wait_gathers(j):
        b = j % NBUF
        t = unit_t(j)

        @pl.when(is_ctx(t))
        def _():
            pltpu.make_async_copy(
                ctx_c.at[gidx_v.at[j, 0]], bufs.at[b, pl.ds(0, HALF)], gsem
            ).wait()
            pltpu.make_async_copy(
                ctx_c.at[gidx_v.at[j, 1]], bufs.at[b, pl.ds(HALF, HALF)], gsem
            ).wait()

        @pl.when(~is_ctx(t))
        def _():
            pltpu.make_async_copy(
                table_c.at[gidx_v.at[j, 0]], bufs.at[b, pl.ds(0, HALF)], gsem
            ).wait()
            pltpu.make_async_copy(
                table_c.at[gidx_v.at[j, 1]], bufs.at[b, pl.ds(HALF, HALF)], gsem
            ).wait()

    def wait_write(j):
        pltpu.make_async_copy(
            bufs.at[j % NBUF], out.at[pl.ds((u0 + j) * UNIT, UNIT)], wsem
        ).wait()

    def body(j, carry):
        @pl.when(j < n_u)
        def _():
            @pl.when(j == 0)
            def _():
                start_gathers(0)

                @pl.when(n_u > 1)
                def _():
                    start_gathers(1)

            wait_gathers(j)

            @pl.when(j + 2 < n_u)
            def _():
                @pl.when(j + 2 >= NBUF)
                def _():
                    wait_write(j + 2 - NBUF)

                start_gathers(j + 2)

            pltpu.async_copy(
                bufs.at[j % NBUF],
                out.at[pl.ds((u0 + j) * UNIT, UNIT)],
                wsem,
            )

        return carry

    lax.fori_loop(0, UPW, body, 0)

    def drain(m, carry):
        j = n_u - NBUF + m

        @pl.when(j >= 0)
        def _():
            wait_write(j)

        return carry

    lax.fori_loop(0, NBUF, drain, 0)


def kernel(token_embedding_weight, ctx, tokenized_prompts):
    table_c = (
        token_embedding_weight.reshape(VOCAB // 8, 8, DB, LANES)
        .transpose(0, 2, 1, 3)
        .reshape(VOCAB * DB, LANES)
    )
    ctx_c = (
        ctx.reshape(1, N_CTX, DB, LANES)
        .transpose(0, 2, 1, 3)
        .reshape(N_CTX * DB, LANES)
    )

    rt = tokenized_prompts.T.reshape(CTX_LEN, KC, CPB // 8, 8)
    base = (rt // 8) * (8 * DB) + (rt % 8)
    dmul = (jnp.arange(DB, dtype=jnp.int32) * 8)[None, None, None, :, None]
    gidx = (base[:, :, :, None, :] + dmul).reshape(CTX_LEN, KC, UNIT)
    cpat = (
        jnp.arange(N_CTX, dtype=jnp.int32)[:, None, None, None]
        + (jnp.arange(DB, dtype=jnp.int32) * 8)[None, None, :, None]
        + jnp.zeros((N_CTX, CPB // 8, DB, 8), jnp.int32)
    ).reshape(N_CTX, UNIT)
    gidx = gidx.at[1:1 + N_CTX].set(cpat[:, None, :])
    gidx = gidx.reshape(UNITS, 2, HALF)
    gidx = jnp.pad(gidx, ((0, UNITS_PAD - UNITS), (0, 0), (0, 0)))

    out = _prompt_gather(table_c, ctx_c, gidx)
    return (
        out.reshape(CTX_LEN, N_CLS // 8, DB, 8, LANES)
        .transpose(1, 3, 0, 2, 4)
        .reshape(N_CLS, CTX_LEN, D)
    )


# speedup vs baseline: 4.9501x; 1.0897x over previous
import functools

import jax
import jax.numpy as jnp
from jax import lax
from jax.experimental import pallas as pl
from jax.experimental.pallas import tpu as pltpu
from jax.experimental.pallas import tpu_sc as plsc

N_CLS = 1000
CTX_LEN = 77
VOCAB = 49408
D = 768
N_CTX = 8
LANES = 128
DB = D // LANES
CPB = 40
KC = N_CLS // CPB
UNIT = CPB * DB
HALF = UNIT // 2
UNITS = CTX_LEN * KC
NW = 32
UPW = (UNITS + NW - 1) // NW
UNITS_PAD = NW * UPW
NBUF = 3

_mesh = plsc.VectorSubcoreMesh(core_axis_name="c", subcore_axis_name="s")


@functools.partial(
    pl.kernel,
    out_type=jax.ShapeDtypeStruct((CTX_LEN * KC * UNIT, LANES), jnp.float32),
    mesh=_mesh,
    scratch_types=[
        pltpu.VMEM((UPW, 2, HALF), jnp.int32),
        pltpu.VMEM((NBUF, UNIT, LANES), jnp.float32),
        pltpu.SemaphoreType.DMA,
        pltpu.SemaphoreType.DMA,
    ],
    compiler_params=pltpu.CompilerParams(use_tc_tiling_on_sc=False),
)
def _prompt_gather(table_c, ctx_c, gidx, out, gidx_v, bufs, gsem, wsem):
    w = lax.axis_index("s") * 2 + lax.axis_index("c")
    n_u = UNITS // NW + jnp.where(w < UNITS % NW, 1, 0)
    pltpu.sync_copy(gidx.at[pl.ds(w * UPW, UPW)], gidx_v)

    def unit_u(j):
        return j * NW + w

    def unit_t(j):
        return unit_u(j) // KC

    def is_ctx(t):
        return (t >= 1) & (t <= 1 + N_CTX - 1)

    def start_gathers(j):
        b = j % NBUF
        t = unit_t(j)

        @pl.when(is_ctx(t))
        def _():
            pltpu.async_copy(
                ctx_c.at[gidx_v.at[j, 0]], bufs.at[b, pl.ds(0, HALF)], gsem)
            pltpu.async_copy(
                ctx_c.at[gidx_v.at[j, 1]], bufs.at[b, pl.ds(HALF, HALF)], gsem)

        @pl.when(~is_ctx(t))
        def _():
            pltpu.async_copy(
                table_c.at[gidx_v.at[j, 0]], bufs.at[b, pl.ds(0, HALF)], gsem)
            pltpu.async_copy(
                table_c.at[gidx_v.at[j, 1]], bufs.at[b, pl.ds(HALF, HALF)], gsem)

    def wait_gathers(j):
        b = j % NBUF
        t = unit_t(j)

        @pl.when(is_ctx(t))
        def _():
            pltpu.make_async_copy(
                ctx_c.at[gidx_v.at[j, 0]], bufs.at[b, pl.ds(0, HALF)], gsem
            ).wait()
            pltpu.make_async_copy(
                ctx_c.at[gidx_v.at[j, 1]], bufs.at[b, pl.ds(HALF, HALF)], gsem
            ).wait()

        @pl.when(~is_ctx(t))
        def _():
            pltpu.make_async_copy(
                table_c.at[gidx_v.at[j, 0]], bufs.at[b, pl.ds(0, HALF)], gsem
            ).wait()
            pltpu.make_async_copy(
                table_c.at[gidx_v.at[j, 1]], bufs.at[b, pl.ds(HALF, HALF)], gsem
            ).wait()

    def wait_write(j):
        pltpu.make_async_copy(
            bufs.at[j % NBUF], out.at[pl.ds(unit_u(j) * UNIT, UNIT)], wsem
        ).wait()

    def body(j, carry):
        @pl.when(j < n_u)
        def _():
            @pl.when(j == 0)
            def _():
                start_gathers(0)

                @pl.when(n_u > 1)
                def _():
                    start_gathers(1)

            wait_gathers(j)

            @pl.when(j + 2 < n_u)
            def _():
                @pl.when(j + 2 >= NBUF)
                def _():
                    wait_write(j + 2 - NBUF)

                start_gathers(j + 2)

            pltpu.async_copy(
                bufs.at[j % NBUF],
                out.at[pl.ds(unit_u(j) * UNIT, UNIT)],
                wsem,
            )

        return carry

    lax.fori_loop(0, UPW, body, 0)

    def drain(m, carry):
        j = n_u - NBUF + m

        @pl.when(j >= 0)
        def _():
            wait_write(j)

        return carry

    lax.fori_loop(0, NBUF, drain, 0)


def kernel(token_embedding_weight, ctx, tokenized_prompts):
    table_c = (
        token_embedding_weight.reshape(VOCAB // 8, 8, DB, LANES)
        .transpose(0, 2, 1, 3)
        .reshape(VOCAB * DB, LANES)
    )
    ctx_c = (
        ctx.reshape(1, N_CTX, DB, LANES)
        .transpose(0, 2, 1, 3)
        .reshape(N_CTX * DB, LANES)
    )

    rt = tokenized_prompts.T.reshape(CTX_LEN, KC, CPB // 8, 8)
    base = (rt // 8) * (8 * DB) + (rt % 8)
    dmul = (jnp.arange(DB, dtype=jnp.int32) * 8)[None, None, None, :, None]
    gidx = (base[:, :, :, None, :] + dmul).reshape(CTX_LEN, KC, UNIT)
    cpat = (
        jnp.arange(N_CTX, dtype=jnp.int32)[:, None, None, None]
        + (jnp.arange(DB, dtype=jnp.int32) * 8)[None, None, :, None]
        + jnp.zeros((N_CTX, CPB // 8, DB, 8), jnp.int32)
    ).reshape(N_CTX, UNIT)
    gidx = gidx.at[1:1 + N_CTX].set(cpat[:, None, :])
    gidx = gidx.reshape(UNITS, 2, HALF)
    gidx = jnp.pad(gidx, ((0, UNITS_PAD - UNITS), (0, 0), (0, 0)))
    gidx = gidx.reshape(UPW, NW, 2, HALF).transpose(1, 0, 2, 3)
    gidx = gidx.reshape(UNITS_PAD, 2, HALF)

    out = _prompt_gather(table_c, ctx_c, gidx)
    return (
        out.reshape(CTX_LEN, N_CLS // 8, DB, 8, LANES)
        .transpose(1, 3, 0, 2, 4)
        .reshape(N_CLS, CTX_LEN, D)
    )


# speedup vs baseline: 7.2121x; 1.4570x over previous
import functools

import jax
import jax.numpy as jnp
from jax import lax
from jax.experimental import pallas as pl
from jax.experimental.pallas import tpu as pltpu
from jax.experimental.pallas import tpu_sc as plsc

N_CLS = 1000
CTX_LEN = 77
VOCAB = 49408
D = 768
N_CTX = 8
LANES = 128
DB = D // LANES
CPB = 40
KC = N_CLS // CPB
UNIT = CPB * DB
HALF = UNIT // 2
UNITS = CTX_LEN * KC
NW = 32
UPW = (UNITS + NW - 1) // NW
UNITS_PAD = NW * UPW
CBLK = 8 * DB
NBUF = 3

_mesh = plsc.VectorSubcoreMesh(core_axis_name="c", subcore_axis_name="s")


@functools.partial(
    pl.kernel,
    out_type=jax.ShapeDtypeStruct((CTX_LEN * KC * UNIT, LANES), jnp.float32),
    mesh=_mesh,
    scratch_types=[
        pltpu.VMEM((UPW, 2, HALF), jnp.int32),
        pltpu.VMEM((NBUF, UNIT, LANES), jnp.float32),
        pltpu.SemaphoreType.DMA,
        pltpu.SemaphoreType.DMA,
    ],
    compiler_params=pltpu.CompilerParams(use_tc_tiling_on_sc=False),
)
def _prompt_gather(table_c, ctx_c, gidx, out, gidx_v, bufs, gsem, wsem):
    w = lax.axis_index("s") * 2 + lax.axis_index("c")
    n_u = UNITS // NW + jnp.where(w < UNITS % NW, 1, 0)
    pltpu.sync_copy(gidx.at[pl.ds(w * UPW, UPW)], gidx_v)

    def unit_u(j):
        return j * NW + w

    def unit_t(j):
        return unit_u(j) // KC

    def is_ctx(t):
        return (t >= 1) & (t <= 1 + N_CTX - 1)

    def start_gathers(j):
        b = j % NBUF
        t = unit_t(j)

        @pl.when(is_ctx(t))
        def _():
            pltpu.async_copy(
                ctx_c.at[gidx_v.at[j, 0, pl.ds(0, CBLK)]],
                bufs.at[b, pl.ds(0, CBLK)], gsem)

        @pl.when(~is_ctx(t))
        def _():
            pltpu.async_copy(
                table_c.at[gidx_v.at[j, 0]], bufs.at[b, pl.ds(0, HALF)], gsem)
            pltpu.async_copy(
                table_c.at[gidx_v.at[j, 1]], bufs.at[b, pl.ds(HALF, HALF)], gsem)

    def wait_gathers(j):
        b = j % NBUF
        t = unit_t(j)

        @pl.when(is_ctx(t))
        def _():
            pltpu.make_async_copy(
                ctx_c.at[gidx_v.at[j, 0, pl.ds(0, CBLK)]],
                bufs.at[b, pl.ds(0, CBLK)], gsem
            ).wait()

        @pl.when(~is_ctx(t))
        def _():
            pltpu.make_async_copy(
                table_c.at[gidx_v.at[j, 0]], bufs.at[b, pl.ds(0, HALF)], gsem
            ).wait()
            pltpu.make_async_copy(
                table_c.at[gidx_v.at[j, 1]], bufs.at[b, pl.ds(HALF, HALF)], gsem
            ).wait()

    def wait_write(j):
        pltpu.make_async_copy(
            bufs.at[j % NBUF], out.at[pl.ds(unit_u(j) * UNIT, UNIT)], wsem
        ).wait()

    def body(j, carry):
        @pl.when(j < n_u)
        def _():
            @pl.when(j == 0)
            def _():
                start_gathers(0)

                @pl.when(n_u > 1)
                def _():
                    start_gathers(1)

            wait_gathers(j)

            @pl.when(j + 2 < n_u)
            def _():
                @pl.when(j + 2 >= NBUF)
                def _():
                    wait_write(j + 2 - NBUF)

                start_gathers(j + 2)

            @pl.when(is_ctx(unit_t(j)))
            def _():
                for m in range(UNIT // CBLK):
                    pltpu.async_copy(
                        bufs.at[j % NBUF, pl.ds(0, CBLK)],
                        out.at[pl.ds(unit_u(j) * UNIT + m * CBLK, CBLK)],
                        wsem,
                    )

            @pl.when(~is_ctx(unit_t(j)))
            def _():
                pltpu.async_copy(
                    bufs.at[j % NBUF],
                    out.at[pl.ds(unit_u(j) * UNIT, UNIT)],
                    wsem,
                )

        return carry

    lax.fori_loop(0, UPW, body, 0)

    def drain(m, carry):
        j = n_u - NBUF + m

        @pl.when(j >= 0)
        def _():
            wait_write(j)

        return carry

    lax.fori_loop(0, NBUF, drain, 0)


def kernel(token_embedding_weight, ctx, tokenized_prompts):
    table_c = (
        token_embedding_weight.reshape(VOCAB // 8, 8, DB, LANES)
        .transpose(0, 2, 1, 3)
        .reshape(VOCAB * DB, LANES)
    )
    ctx_c = (
        ctx.reshape(1, N_CTX, DB, LANES)
        .transpose(0, 2, 1, 3)
        .reshape(N_CTX * DB, LANES)
    )

    rt = tokenized_prompts.T.reshape(CTX_LEN, KC, CPB // 8, 8)
    base = (rt // 8) * (8 * DB) + (rt % 8)
    dmul = (jnp.arange(DB, dtype=jnp.int32) * 8)[None, None, None, :, None]
    gidx = (base[:, :, :, None, :] + dmul).reshape(CTX_LEN, KC, UNIT)
    cpat = (
        jnp.arange(N_CTX, dtype=jnp.int32)[:, None, None, None]
        + (jnp.arange(DB, dtype=jnp.int32) * 8)[None, None, :, None]
        + jnp.zeros((N_CTX, CPB // 8, DB, 8), jnp.int32)
    ).reshape(N_CTX, UNIT)
    gidx = gidx.at[1:1 + N_CTX].set(cpat[:, None, :])
    gidx = gidx.reshape(UNITS, 2, HALF)
    gidx = jnp.pad(gidx, ((0, UNITS_PAD - UNITS), (0, 0), (0, 0)))
    gidx = gidx.reshape(UPW, NW, 2, HALF).transpose(1, 0, 2, 3)
    gidx = gidx.reshape(UNITS_PAD, 2, HALF)

    out = _prompt_gather(table_c, ctx_c, gidx)
    return (
        out.reshape(CTX_LEN, N_CLS // 8, DB, 8, LANES)
        .transpose(1, 3, 0, 2, 4)
        .reshape(N_CLS, CTX_LEN, D)
    )


# speedup vs baseline: 7.2152x; 1.0004x over previous
import functools

import jax
import jax.numpy as jnp
from jax import lax
from jax.experimental import pallas as pl
from jax.experimental.pallas import tpu as pltpu
from jax.experimental.pallas import tpu_sc as plsc

N_CLS = 1000
CTX_LEN = 77
VOCAB = 49408
D = 768
N_CTX = 8
LANES = 128
DB = D // LANES
CPB = 40
KC = N_CLS // CPB
UNIT = CPB * DB
HALF = UNIT // 2
UNITS = CTX_LEN * KC
NW = 32
UPW = (UNITS + NW - 1) // NW
UNITS_PAD = NW * UPW
CBLK = 8 * DB
NBUF = 4
PD = 3

_mesh = plsc.VectorSubcoreMesh(core_axis_name="c", subcore_axis_name="s")


@functools.partial(
    pl.kernel,
    out_type=jax.ShapeDtypeStruct((CTX_LEN * KC * UNIT, LANES), jnp.float32),
    mesh=_mesh,
    scratch_types=[
        pltpu.VMEM((NBUF, 2, HALF), jnp.int32),
        pltpu.VMEM((NBUF, UNIT, LANES), jnp.float32),
        pltpu.SemaphoreType.DMA,
        pltpu.SemaphoreType.DMA,
        pltpu.SemaphoreType.DMA,
    ],
    compiler_params=pltpu.CompilerParams(use_tc_tiling_on_sc=False),
)
def _prompt_gather(table_c, ctx_c, gidx, out, gidx_v, bufs, gsem, wsem, isem):
    w = lax.axis_index("s") * 2 + lax.axis_index("c")
    n_u = UNITS // NW + jnp.where(w < UNITS % NW, 1, 0)

    def unit_u(j):
        return j * NW + w

    def unit_t(j):
        return unit_u(j) // KC

    def is_ctx(t):
        return (t >= 1) & (t <= N_CTX)

    def start_idx(j):
        pltpu.async_copy(gidx.at[w * UPW + j], gidx_v.at[j % NBUF], isem)

    def wait_idx(j):
        pltpu.make_async_copy(
            gidx.at[w * UPW + j], gidx_v.at[j % NBUF], isem
        ).wait()

    def start_gathers(j):
        b = j % NBUF
        t = unit_t(j)

        @pl.when(is_ctx(t))
        def _():
            pltpu.async_copy(
                ctx_c.at[gidx_v.at[b, 0, pl.ds(0, CBLK)]],
                bufs.at[b, pl.ds(0, CBLK)], gsem)

        @pl.when(~is_ctx(t))
        def _():
            pltpu.async_copy(
                table_c.at[gidx_v.at[b, 0]], bufs.at[b, pl.ds(0, HALF)], gsem)
            pltpu.async_copy(
                table_c.at[gidx_v.at[b, 1]], bufs.at[b, pl.ds(HALF, HALF)],
                gsem)

    def wait_gathers(j):
        b = j % NBUF
        t = unit_t(j)

        @pl.when(is_ctx(t))
        def _():
            pltpu.make_async_copy(
                ctx_c.at[gidx_v.at[b, 0, pl.ds(0, CBLK)]],
                bufs.at[b, pl.ds(0, CBLK)], gsem
            ).wait()

        @pl.when(~is_ctx(t))
        def _():
            pltpu.make_async_copy(
                table_c.at[gidx_v.at[b, 0]], bufs.at[b, pl.ds(0, HALF)], gsem
            ).wait()
            pltpu.make_async_copy(
                table_c.at[gidx_v.at[b, 1]], bufs.at[b, pl.ds(HALF, HALF)],
                gsem
            ).wait()

    def wait_write(j):
        pltpu.make_async_copy(
            bufs.at[j % NBUF], out.at[pl.ds(unit_u(j) * UNIT, UNIT)], wsem
        ).wait()

    def body(j, carry):
        @pl.when(j < n_u)
        def _():
            @pl.when(j == 0)
            def _():
                for jj in range(PD):
                    @pl.when(jj < n_u)
                    def _(jj=jj):
                        start_idx(jj)

                wait_idx(0)
                start_gathers(0)

                @pl.when(n_u > 1)
                def _():
                    wait_idx(1)
                    start_gathers(1)

            wait_gathers(j)

            @pl.when(is_ctx(unit_t(j)))
            def _():
                for m in range(UNIT // CBLK):
                    pltpu.async_copy(
                        bufs.at[j % NBUF, pl.ds(0, CBLK)],
                        out.at[pl.ds(unit_u(j) * UNIT + m * CBLK, CBLK)],
                        wsem,
                    )

            @pl.when(~is_ctx(unit_t(j)))
            def _():
                pltpu.async_copy(
                    bufs.at[j % NBUF],
                    out.at[pl.ds(unit_u(j) * UNIT, UNIT)],
                    wsem,
                )

            @pl.when(j + 2 < n_u)
            def _():
                @pl.when(j + 2 >= NBUF)
                def _():
                    wait_write(j + 2 - NBUF)

                wait_idx(j + 2)
                start_gathers(j + 2)

            @pl.when(j + PD < n_u)
            def _():
                start_idx(j + PD)

        return carry

    lax.fori_loop(0, UPW, body, 0)

    def drain(m, carry):
        j = n_u - NBUF + m

        @pl.when(j >= 0)
        def _():
            wait_write(j)

        return carry

    lax.fori_loop(0, NBUF, drain, 0)


def kernel(token_embedding_weight, ctx, tokenized_prompts):
    table_c = (
        token_embedding_weight.reshape(VOCAB // 8, 8, DB, LANES)
        .transpose(0, 2, 1, 3)
        .reshape(VOCAB * DB, LANES)
    )
    ctx_c = (
        ctx.reshape(1, N_CTX, DB, LANES)
        .transpose(0, 2, 1, 3)
        .reshape(N_CTX * DB, LANES)
    )

    rt = tokenized_prompts.T.reshape(CTX_LEN, KC, CPB // 8, 8)
    base = (rt // 8) * (8 * DB) + (rt % 8)
    dmul = (jnp.arange(DB, dtype=jnp.int32) * 8)[None, None, None, :, None]
    gidx = (base[:, :, :, None, :] + dmul).reshape(CTX_LEN, KC, UNIT)
    cpat = (
        jnp.arange(N_CTX, dtype=jnp.int32)[:, None, None, None]
        + (jnp.arange(DB, dtype=jnp.int32) * 8)[None, None, :, None]
        + jnp.zeros((N_CTX, CPB // 8, DB, 8), jnp.int32)
    ).reshape(N_CTX, UNIT)
    gidx = gidx.at[1:1 + N_CTX].set(cpat[:, None, :])
    gidx = gidx.reshape(UNITS, 2, HALF)
    gidx = jnp.pad(gidx, ((0, UNITS_PAD - UNITS), (0, 0), (0, 0)))
    gidx = gidx.reshape(UPW, NW, 2, HALF).transpose(1, 0, 2, 3)
    gidx = gidx.reshape(UNITS_PAD, 2, HALF)

    out = _prompt_gather(table_c, ctx_c, gidx)
    return (
        out.reshape(CTX_LEN, N_CLS // 8, DB, 8, LANES)
        .transpose(1, 3, 0, 2, 4)
        .reshape(N_CLS, CTX_LEN, D)
    )
